# baseline (speedup 1.0000x reference)
import functools

import jax
import jax.numpy as jnp
from jax import lax
from jax.experimental import pallas as pl
from jax.experimental.pallas import tpu as pltpu
from jax.experimental.pallas import tpu_sc as plsc

NUM_CORES = 2
NUM_SUBCORES = 16
NW = NUM_CORES * NUM_SUBCORES

CHUNK = 128
NBUF = 4
D = 128


def _sc_gather(n_rows, n_blocks):
    per_w = n_rows // NW
    assert n_blocks % 2 == 0 and n_blocks >= 4
    mesh = plsc.VectorSubcoreMesh(core_axis_name="c", subcore_axis_name="s")

    @functools.partial(
        pl.kernel,
        mesh=mesh,
        out_type=jax.ShapeDtypeStruct((n_rows, D), jnp.float32),
        scratch_types=[
            pltpu.VMEM((2, NBUF, CHUNK), jnp.int32),
            pltpu.VMEM((NBUF, CHUNK, D), jnp.float32),
            pltpu.SemaphoreType.DMA((2,)),
            pltpu.SemaphoreType.DMA((NBUF,)),
            pltpu.SemaphoreType.DMA((NBUF,)),
        ],
    )
    def body(x_hbm, table_hbm, out_hbm, idx_v, rows_v, isem, gsem, wsem):
        wid = lax.axis_index("s") * NUM_CORES + lax.axis_index("c")
        row_base = wid * per_w

        def gather(par, b):
            return pltpu.make_async_copy(
                table_hbm.at[idx_v.at[par, b]], rows_v.at[b], gsem.at[b]
            )

        def write(t, b):
            row0 = row_base + (t * NBUF + b) * CHUNK
            return pltpu.make_async_copy(
                rows_v.at[b], out_hbm.at[pl.ds(row0, CHUNK)], wsem.at[b]
            )

        def idx_load(par, blk):
            return pltpu.make_async_copy(
                x_hbm.at[wid, blk], idx_v.at[par], isem.at[par]
            )

        def do_block(t, par, wait_idx, issue_gather, issue_idx):
            p1 = 1 - par
            if wait_idx:
                idx_load(p1, t + 1).wait()
            for b in range(NBUF):
                gather(par, b).wait()
                write(t, b).start()
                if issue_gather and b > 0:
                    write(t, b - 1).wait()
                    gather(p1, b - 1).start()
            if issue_gather:
                write(t, NBUF - 1).wait()
                gather(p1, NBUF - 1).start()
            else:
                for b in range(NBUF):
                    write(t, b).wait()
            if issue_idx:
                idx_load(par, t + 2).start()

        pltpu.sync_copy(x_hbm.at[wid, 0], idx_v.at[0])
        idx_load(1, 1).start()
        for b in range(NBUF):
            gather(0, b).start()

        def outer(s, carry):
            do_block(2 * s, 0, True, True, True)
            do_block(2 * s + 1, 1, True, True, True)
            return carry

        lax.fori_loop(0, (n_blocks - 2) // 2, outer, 0)
        do_block(n_blocks - 2, (n_blocks - 2) % 2, True, True, False)
        do_block(n_blocks - 1, (n_blocks - 1) % 2, False, False, False)

    return body


def kernel(x, table):
    n_rows = x.size
    n_blocks = n_rows // (NW * NBUF * CHUNK)
    x_r = x.reshape(NW, n_blocks, NBUF, CHUNK).astype(jnp.int32)
    out = _sc_gather(n_rows, n_blocks)(x_r, table)
    return out.reshape(*x.shape, D)


# speedup vs baseline: 1.0058x; 1.0058x over previous
import functools

import jax
import jax.numpy as jnp
from jax import lax
from jax.experimental import pallas as pl
from jax.experimental.pallas import tpu as pltpu
from jax.experimental.pallas import tpu_sc as plsc

NUM_CORES = 2
NUM_SUBCORES = 16
NW = NUM_CORES * NUM_SUBCORES

CHUNK = 128
NBUF = 5
D = 128


def _sc_gather(n_rows, n_blocks):
    per_w = n_rows // NW
    assert n_blocks % 2 == 0 and n_blocks >= 4
    mesh = plsc.VectorSubcoreMesh(core_axis_name="c", subcore_axis_name="s")

    @functools.partial(
        pl.kernel,
        mesh=mesh,
        out_type=jax.ShapeDtypeStruct((n_rows, D), jnp.float32),
        scratch_types=[
            pltpu.VMEM((2, NBUF, CHUNK), jnp.int32),
            pltpu.VMEM((NBUF, CHUNK, D), jnp.float32),
            pltpu.SemaphoreType.DMA((2,)),
            pltpu.SemaphoreType.DMA((NBUF,)),
            pltpu.SemaphoreType.DMA((NBUF,)),
        ],
    )
    def body(x_hbm, table_hbm, out_hbm, idx_v, rows_v, isem, gsem, wsem):
        wid = lax.axis_index("s") * NUM_CORES + lax.axis_index("c")
        row_base = wid * per_w

        def gather(par, b):
            return pltpu.make_async_copy(
                table_hbm.at[idx_v.at[par, b]], rows_v.at[b], gsem.at[b]
            )

        def write(t, b):
            row0 = row_base + (t * NBUF + b) * CHUNK
            return pltpu.make_async_copy(
                rows_v.at[b], out_hbm.at[pl.ds(row0, CHUNK)], wsem.at[b]
            )

        def idx_load(par, blk):
            return pltpu.make_async_copy(
                x_hbm.at[wid, blk], idx_v.at[par], isem.at[par]
            )

        def do_block(t, par, wait_idx, issue_gather, issue_idx):
            p1 = 1 - par
            if wait_idx:
                idx_load(p1, t + 1).wait()
            for b in range(NBUF):
                gather(par, b).wait()
                write(t, b).start()
                if issue_gather and b > 0:
                    write(t, b - 1).wait()
                    gather(p1, b - 1).start()
            if issue_gather:
                write(t, NBUF - 1).wait()
                gather(p1, NBUF - 1).start()
            else:
                for b in range(NBUF):
                    write(t, b).wait()
            if issue_idx:
                idx_load(par, t + 2).start()

        pltpu.sync_copy(x_hbm.at[wid, 0], idx_v.at[0])
        idx_load(1, 1).start()
        for b in range(NBUF):
            gather(0, b).start()

        def outer(s, carry):
            do_block(2 * s, 0, True, True, True)
            do_block(2 * s + 1, 1, True, True, True)
            return carry

        lax.fori_loop(0, (n_blocks - 2) // 2, outer, 0)
        do_block(n_blocks - 2, (n_blocks - 2) % 2, True, True, False)
        do_block(n_blocks - 1, (n_blocks - 1) % 2, False, False, False)

    return body


def kernel(x, table):
    n_rows = x.size
    n_blocks = n_rows // (NW * NBUF * CHUNK)
    x_r = x.reshape(NW, n_blocks, NBUF, CHUNK).astype(jnp.int32)
    out = _sc_gather(n_rows, n_blocks)(x_r, table)
    return out.reshape(*x.shape, D)
